# baseline (speedup 1.0000x reference)
import jax
import jax.numpy as jnp
from jax.experimental import pallas as pl
from jax.experimental.pallas import tpu as pltpu

_TOPK = 8
_NE = 64
_BLOCK = 512


def _gate_kernel(x_ref, w_ref, b_ref, wts_ref, idx_ref):
    x = x_ref[...]
    w = w_ref[...]
    logits = jax.lax.dot_general(
        x, w, (((1,), (1,)), ((), ())), preferred_element_type=jnp.float32
    )
    logits = logits + b_ref[...]

    bits = jax.lax.bitcast_convert_type(logits, jnp.int32)
    srt = jnp.where(bits >= 0, bits, bits ^ jnp.int32(0x7FFFFFFF))
    cols = jax.lax.broadcasted_iota(jnp.int32, logits.shape, 1)
    key = (srt & jnp.int32(-64)) | (jnp.int32(_NE - 1) - cols)

    int_min = jnp.int32(-(2**31))
    tops = []
    for _ in range(_TOPK):
        m = jnp.max(key, axis=-1, keepdims=True)
        tops.append(m)
        key = jnp.where(key == m, int_min, key)

    mk = jnp.concatenate(tops, axis=-1)
    tidx = jnp.int32(_NE - 1) - (mk & jnp.int32(_NE - 1))
    sv = mk & jnp.int32(-64)
    fbits = jnp.where(sv >= 0, sv, sv ^ jnp.int32(0x7FFFFFFF))
    top = jax.lax.bitcast_convert_type(fbits, jnp.float32)

    e = jnp.exp(top - top[:, 0:1])
    wts_ref[...] = e / jnp.sum(e, axis=-1, keepdims=True)
    idx_ref[...] = tidx


def kernel(x, W, b):
    n, d = x.shape
    grid = (n // _BLOCK,)
    wts, idx = pl.pallas_call(
        _gate_kernel,
        grid=grid,
        in_specs=[
            pl.BlockSpec((_BLOCK, d), lambda i: (i, 0)),
            pl.BlockSpec((_NE, d), lambda i: (0, 0)),
            pl.BlockSpec((1, _NE), lambda i: (0, 0)),
        ],
        out_specs=[
            pl.BlockSpec((_BLOCK, _TOPK), lambda i: (i, 0)),
            pl.BlockSpec((_BLOCK, _TOPK), lambda i: (i, 0)),
        ],
        out_shape=[
            jax.ShapeDtypeStruct((n, _TOPK), jnp.float32),
            jax.ShapeDtypeStruct((n, _TOPK), jnp.int32),
        ],
        compiler_params=pltpu.CompilerParams(
            dimension_semantics=("parallel",),
        ),
    )(x, W, b.reshape(1, _NE))
    return wts, idx


# speedup vs baseline: 1.1167x; 1.1167x over previous
import jax
import jax.numpy as jnp
from jax.experimental import pallas as pl
from jax.experimental.pallas import tpu as pltpu

_TOPK = 8
_NE = 64
_BLOCK = 512


def _gate_kernel(x_ref, w_ref, b_ref, wts_ref, idx_ref):
    x = x_ref[...]
    w = w_ref[...]
    logits = jax.lax.dot_general(
        x, w, (((1,), (1,)), ((), ())), preferred_element_type=jnp.float32
    )
    logits = logits + b_ref[...]

    bits = jax.lax.bitcast_convert_type(logits, jnp.int32)
    cols = jax.lax.broadcasted_iota(jnp.int32, logits.shape, 1)
    code = jnp.where(bits >= 0, jnp.int32(_NE - 1) - cols, cols)
    key = jax.lax.bitcast_convert_type(
        (bits & jnp.int32(-64)) | code, jnp.float32
    )

    neg_inf = jnp.float32(-jnp.inf)
    tops = []
    for _ in range(_TOPK):
        m = jnp.max(key, axis=-1, keepdims=True)
        tops.append(m)
        key = jnp.where(key == m, neg_inf, key)

    mf = jnp.concatenate(tops, axis=-1)
    mb = jax.lax.bitcast_convert_type(mf, jnp.int32)
    low = mb & jnp.int32(_NE - 1)
    tidx = jnp.where(mb >= 0, jnp.int32(_NE - 1) - low, low)
    top = jax.lax.bitcast_convert_type(mb & jnp.int32(-64), jnp.float32)

    e = jnp.exp(top - top[:, 0:1])
    wts_ref[...] = e / jnp.sum(e, axis=-1, keepdims=True)
    idx_ref[...] = tidx


def kernel(x, W, b):
    n, d = x.shape
    grid = (n // _BLOCK,)
    wts, idx = pl.pallas_call(
        _gate_kernel,
        grid=grid,
        in_specs=[
            pl.BlockSpec((_BLOCK, d), lambda i: (i, 0)),
            pl.BlockSpec((_NE, d), lambda i: (0, 0)),
            pl.BlockSpec((1, _NE), lambda i: (0, 0)),
        ],
        out_specs=[
            pl.BlockSpec((_BLOCK, _TOPK), lambda i: (i, 0)),
            pl.BlockSpec((_BLOCK, _TOPK), lambda i: (i, 0)),
        ],
        out_shape=[
            jax.ShapeDtypeStruct((n, _TOPK), jnp.float32),
            jax.ShapeDtypeStruct((n, _TOPK), jnp.int32),
        ],
        compiler_params=pltpu.CompilerParams(
            dimension_semantics=("parallel",),
        ),
    )(x, W, b.reshape(1, _NE))
    return wts, idx
